# baseline (speedup 1.0000x reference)
import functools

import jax
import jax.numpy as jnp
from jax import lax
from jax.experimental import pallas as pl
from jax.experimental.pallas import tpu as pltpu
from jax.experimental.pallas import tpu_sc as plsc

N = 10000
E = 320000
DF = 128
DH = 256
DC = 16

NC = 2
NS = 16
NW = NC * NS
EW = E // NW
RT = N // NS


def _make_sc_agg(D, C, S, count):
  NSC = EW // (S * C)
  assert NSC * S * C == EW and C <= 128
  mesh = plsc.VectorSubcoreMesh(core_axis_name="c", subcore_axis_name="s")

  out_type = [jax.ShapeDtypeStruct((NC, N, D), jnp.float32)]
  scratch = [
      pltpu.VMEM((S, C), jnp.int32),
      pltpu.VMEM((S, C), jnp.int32),
      pltpu.VMEM((C, D), jnp.float32),
      pltpu.VMEM((C, D), jnp.float32),
      pltpu.VMEM_SHARED((N, D), jnp.float32),
      pltpu.SemaphoreType.DMA,
      pltpu.SemaphoreType.DMA,
  ]
  if count:
    out_type.append(jax.ShapeDtypeStruct((NC, N, DC), jnp.float32))
    scratch += [
        pltpu.VMEM((C, DC), jnp.float32),
        pltpu.VMEM_SHARED((N, DC), jnp.float32),
    ]

  @functools.partial(pl.kernel,
                     out_type=out_type if count else out_type[0], mesh=mesh,
                     compiler_params=pltpu.CompilerParams(
                         use_tc_tiling_on_sc=False),
                     scratch_types=scratch)
  def sc_agg(table, srcr, dstr, zrow, *rest):
    if count:
      (zcnt, out, outc, isrc, idst, rows0, rows1, acc, sem0, sem1,
       ones, cacc) = rest
    else:
      out, isrc, idst, rows0, rows1, acc, sem0, sem1 = rest
    c = lax.axis_index("c")
    s = lax.axis_index("s")
    w = s * NC + c
    base = s * RT

    pltpu.sync_copy(zrow, acc.at[pl.ds(base, RT)])
    if count:
      pltpu.sync_copy(zcnt, cacc.at[pl.ds(base, RT)])

      @pl.loop(0, C)
      def _fill(i):
        ones[i] = jnp.ones((DC,), jnp.float32)

    plsc.subcore_barrier()

    def scatter(rows, k):
      pltpu.sync_copy(rows, acc.at[idst.at[k]], add=True)
      if count:
        pltpu.sync_copy(ones, cacc.at[idst.at[k]], add=True)

    @pl.loop(0, NSC)
    def _super(u):
      pltpu.sync_copy(srcr.at[w].at[u], isrc)
      pltpu.sync_copy(dstr.at[w].at[u], idst)
      pltpu.async_copy(table.at[isrc.at[0]], rows0, sem0)

      @pl.loop(0, S - 1, step=2)
      def _edges(k):
        pltpu.async_copy(table.at[isrc.at[k + 1]], rows1, sem1)
        pltpu.make_async_copy(table.at[isrc.at[k]], rows0, sem0).wait()
        scatter(rows0, k)

        @pl.when(k + 2 < S)
        def _():
          pltpu.async_copy(table.at[isrc.at[k + 2]], rows0, sem0)

        pltpu.make_async_copy(table.at[isrc.at[k + 1]], rows1, sem1).wait()
        scatter(rows1, k + 1)

      if S % 2:
        pltpu.make_async_copy(table.at[isrc.at[S - 1]], rows0, sem0).wait()
        scatter(rows0, S - 1)

    plsc.subcore_barrier()

    pltpu.sync_copy(acc.at[pl.ds(base, RT)], out.at[c].at[pl.ds(base, RT)])
    if count:
      pltpu.sync_copy(cacc.at[pl.ds(base, RT)], outc.at[c].at[pl.ds(base, RT)])

  return sc_agg


CA, SA = 80, 25
CF, SF = 80, 25
_sc_agg_c = _make_sc_agg(DF, CA, SA, True)
_sc_agg_f = _make_sc_agg(DF, CF, SF, False)

BN = 1000
GRID = N // BN


def _tc1_body(aggp, cntp, x, w1l, w1r, b1, w2l, w2r, b2, g, hr, inv):
  agg = aggp[0] + aggp[1]
  cnt = (cntp[0] + cntp[1])[:, :1]
  iv = 1.0 / jnp.maximum(cnt, 1.0)
  mean = agg * iv
  h = (jnp.dot(mean, w1l[...], preferred_element_type=jnp.float32)
       + jnp.dot(x[...], w1r[...], preferred_element_type=jnp.float32)
       + b1[...])
  h = jnp.maximum(h, 0.0)
  g[...] = jnp.dot(h, w2l[...], preferred_element_type=jnp.float32)
  hr[...] = (jnp.dot(h, w2r[...], preferred_element_type=jnp.float32)
             + b2[...])
  inv[...] = iv


_tc1 = pl.pallas_call(
    _tc1_body,
    grid=(GRID,),
    in_specs=[
        pl.BlockSpec((NC, BN, DF), lambda i: (0, i, 0)),
        pl.BlockSpec((NC, BN, DC), lambda i: (0, i, 0)),
        pl.BlockSpec((BN, DF), lambda i: (i, 0)),
        pl.BlockSpec((DF, DH), lambda i: (0, 0)),
        pl.BlockSpec((DF, DH), lambda i: (0, 0)),
        pl.BlockSpec((1, DH), lambda i: (0, 0)),
        pl.BlockSpec((DH, DF), lambda i: (0, 0)),
        pl.BlockSpec((DH, DF), lambda i: (0, 0)),
        pl.BlockSpec((1, DF), lambda i: (0, 0)),
    ],
    out_specs=[
        pl.BlockSpec((BN, DF), lambda i: (i, 0)),
        pl.BlockSpec((BN, DF), lambda i: (i, 0)),
        pl.BlockSpec((BN, 1), lambda i: (i, 0)),
    ],
    out_shape=[
        jax.ShapeDtypeStruct((N, DF), jnp.float32),
        jax.ShapeDtypeStruct((N, DF), jnp.float32),
        jax.ShapeDtypeStruct((N, 1), jnp.float32),
    ],
)


def _tc2_body(agg2, inv, hr, out):
  out[...] = (agg2[0] + agg2[1]) * inv[...] + hr[...]


_tc2 = pl.pallas_call(
    _tc2_body,
    grid=(GRID,),
    in_specs=[
        pl.BlockSpec((NC, BN, DF), lambda i: (0, i, 0)),
        pl.BlockSpec((BN, 1), lambda i: (i, 0)),
        pl.BlockSpec((BN, DF), lambda i: (i, 0)),
    ],
    out_specs=pl.BlockSpec((BN, DF), lambda i: (i, 0)),
    out_shape=jax.ShapeDtypeStruct((N, DF), jnp.float32),
)


def kernel(x, edge_index, W1_l, W1_r, b1, W2_l, W2_r, b2):
  src = edge_index[0].astype(jnp.int32)
  dst = edge_index[1].astype(jnp.int32)
  src_a = src.reshape(NW, EW // (SA * CA), SA, CA)
  dst_a = dst.reshape(NW, EW // (SA * CA), SA, CA)
  src_f = src.reshape(NW, EW // (SF * CF), SF, CF)
  dst_f = dst.reshape(NW, EW // (SF * CF), SF, CF)
  zrow = jnp.zeros((RT, DF), jnp.float32)
  zcnt = jnp.zeros((RT, DC), jnp.float32)

  aggp, cntp = _sc_agg_c(x, src_a, dst_a, zrow, zcnt)
  g, hr, inv = _tc1(aggp, cntp, x, W1_l, W1_r, b1.reshape(1, DH),
                    W2_l, W2_r, b2.reshape(1, DF))
  agg2 = _sc_agg_f(g, src_f, dst_f, zrow)
  return _tc2(agg2, inv, hr)


# speedup vs baseline: 1.0623x; 1.0623x over previous
import functools

import jax
import jax.numpy as jnp
from jax import lax
from jax.experimental import pallas as pl
from jax.experimental.pallas import tpu as pltpu
from jax.experimental.pallas import tpu_sc as plsc

N = 10000
E = 320000
DF = 128
DH = 256
DC = 16

NC = 2
NS = 16
NW = NC * NS
EW = E // NW
RT = N // NS


def _make_sc_agg(D, C, S, count, dtype=jnp.float32):
  NSC = EW // (S * C)
  assert NSC * S * C == EW and C <= 128
  mesh = plsc.VectorSubcoreMesh(core_axis_name="c", subcore_axis_name="s")

  out_type = [jax.ShapeDtypeStruct((NC, N, D), dtype)]
  scratch = [
      pltpu.VMEM((S, C), jnp.int32),
      pltpu.VMEM((S, C), jnp.int32),
      pltpu.VMEM((C, D), dtype),
      pltpu.VMEM((C, D), dtype),
      pltpu.VMEM_SHARED((N, D), dtype),
      pltpu.SemaphoreType.DMA,
      pltpu.SemaphoreType.DMA,
  ]
  if count:
    out_type.append(jax.ShapeDtypeStruct((NC, N, DC), jnp.float32))
    scratch += [
        pltpu.VMEM((C, DC), jnp.float32),
        pltpu.VMEM_SHARED((N, DC), jnp.float32),
    ]

  @functools.partial(pl.kernel,
                     out_type=out_type if count else out_type[0], mesh=mesh,
                     compiler_params=pltpu.CompilerParams(
                         use_tc_tiling_on_sc=False),
                     scratch_types=scratch)
  def sc_agg(table, srcr, dstr, zrow, *rest):
    if count:
      (zcnt, out, outc, isrc, idst, rows0, rows1, acc, sem0, sem1,
       ones, cacc) = rest
    else:
      out, isrc, idst, rows0, rows1, acc, sem0, sem1 = rest
    c = lax.axis_index("c")
    s = lax.axis_index("s")
    w = s * NC + c
    base = s * RT

    pltpu.sync_copy(zrow, acc.at[pl.ds(base, RT)])
    if count:
      pltpu.sync_copy(zcnt, cacc.at[pl.ds(base, RT)])

      @pl.loop(0, C)
      def _fill(i):
        ones[i] = jnp.ones((DC,), jnp.float32)

    plsc.subcore_barrier()

    def scatter(rows, k):
      pltpu.sync_copy(rows, acc.at[idst.at[k]], add=True)
      if count:
        pltpu.sync_copy(ones, cacc.at[idst.at[k]], add=True)

    @pl.loop(0, NSC)
    def _super(u):
      pltpu.sync_copy(srcr.at[w].at[u], isrc)
      pltpu.sync_copy(dstr.at[w].at[u], idst)
      pltpu.async_copy(table.at[isrc.at[0]], rows0, sem0)

      @pl.loop(0, S - 1, step=2)
      def _edges(k):
        pltpu.async_copy(table.at[isrc.at[k + 1]], rows1, sem1)
        pltpu.make_async_copy(table.at[isrc.at[k]], rows0, sem0).wait()
        scatter(rows0, k)

        @pl.when(k + 2 < S)
        def _():
          pltpu.async_copy(table.at[isrc.at[k + 2]], rows0, sem0)

        pltpu.make_async_copy(table.at[isrc.at[k + 1]], rows1, sem1).wait()
        scatter(rows1, k + 1)

      if S % 2:
        pltpu.make_async_copy(table.at[isrc.at[S - 1]], rows0, sem0).wait()
        scatter(rows0, S - 1)

    plsc.subcore_barrier()

    pltpu.sync_copy(acc.at[pl.ds(base, RT)], out.at[c].at[pl.ds(base, RT)])
    if count:
      pltpu.sync_copy(cacc.at[pl.ds(base, RT)], outc.at[c].at[pl.ds(base, RT)])

  return sc_agg


CA, SA = 80, 25
CF, SF = 80, 25
_sc_agg_c = _make_sc_agg(DF, CA, SA, True, jnp.bfloat16)
_sc_agg_f = _make_sc_agg(DF, CF, SF, False, jnp.bfloat16)

BN = 1000
GRID = N // BN


def _tc1_body(aggp, cntp, x, w1l, w1r, b1, w2l, w2r, b2, g, hr, inv):
  agg = (aggp[0].astype(jnp.float32) + aggp[1].astype(jnp.float32))
  cnt = (cntp[0] + cntp[1])[:, :1]
  iv = 1.0 / jnp.maximum(cnt, 1.0)
  mean = agg * iv
  h = (jnp.dot(mean, w1l[...], preferred_element_type=jnp.float32)
       + jnp.dot(x[...], w1r[...], preferred_element_type=jnp.float32)
       + b1[...])
  h = jnp.maximum(h, 0.0)
  g[...] = jnp.dot(h, w2l[...],
                   preferred_element_type=jnp.float32).astype(jnp.bfloat16)
  hr[...] = (jnp.dot(h, w2r[...], preferred_element_type=jnp.float32)
             + b2[...])
  inv[...] = iv


_tc1 = pl.pallas_call(
    _tc1_body,
    grid=(GRID,),
    in_specs=[
        pl.BlockSpec((NC, BN, DF), lambda i: (0, i, 0)),
        pl.BlockSpec((NC, BN, DC), lambda i: (0, i, 0)),
        pl.BlockSpec((BN, DF), lambda i: (i, 0)),
        pl.BlockSpec((DF, DH), lambda i: (0, 0)),
        pl.BlockSpec((DF, DH), lambda i: (0, 0)),
        pl.BlockSpec((1, DH), lambda i: (0, 0)),
        pl.BlockSpec((DH, DF), lambda i: (0, 0)),
        pl.BlockSpec((DH, DF), lambda i: (0, 0)),
        pl.BlockSpec((1, DF), lambda i: (0, 0)),
    ],
    out_specs=[
        pl.BlockSpec((BN, DF), lambda i: (i, 0)),
        pl.BlockSpec((BN, DF), lambda i: (i, 0)),
        pl.BlockSpec((BN, 1), lambda i: (i, 0)),
    ],
    out_shape=[
        jax.ShapeDtypeStruct((N, DF), jnp.bfloat16),
        jax.ShapeDtypeStruct((N, DF), jnp.float32),
        jax.ShapeDtypeStruct((N, 1), jnp.float32),
    ],
)


def _tc2_body(agg2, inv, hr, out):
  s = agg2[0].astype(jnp.float32) + agg2[1].astype(jnp.float32)
  out[...] = s * inv[...] + hr[...]


_tc2 = pl.pallas_call(
    _tc2_body,
    grid=(GRID,),
    in_specs=[
        pl.BlockSpec((NC, BN, DF), lambda i: (0, i, 0)),
        pl.BlockSpec((BN, 1), lambda i: (i, 0)),
        pl.BlockSpec((BN, DF), lambda i: (i, 0)),
    ],
    out_specs=pl.BlockSpec((BN, DF), lambda i: (i, 0)),
    out_shape=jax.ShapeDtypeStruct((N, DF), jnp.float32),
)


def kernel(x, edge_index, W1_l, W1_r, b1, W2_l, W2_r, b2):
  src = edge_index[0].astype(jnp.int32)
  dst = edge_index[1].astype(jnp.int32)
  src_a = src.reshape(NW, EW // (SA * CA), SA, CA)
  dst_a = dst.reshape(NW, EW // (SA * CA), SA, CA)
  src_f = src.reshape(NW, EW // (SF * CF), SF, CF)
  dst_f = dst.reshape(NW, EW // (SF * CF), SF, CF)
  zrow = jnp.zeros((RT, DF), jnp.bfloat16)
  zcnt = jnp.zeros((RT, DC), jnp.float32)
  xb = x.astype(jnp.bfloat16)

  aggp, cntp = _sc_agg_c(xb, src_a, dst_a, zrow, zcnt)
  g, hr, inv = _tc1(aggp, cntp, x, W1_l, W1_r, b1.reshape(1, DH),
                    W2_l, W2_r, b2.reshape(1, DF))
  agg2 = _sc_agg_f(g, src_f, dst_f, zrow)
  return _tc2(agg2, inv, hr)


# speedup vs baseline: 1.2210x; 1.1494x over previous
import functools

import jax
import jax.numpy as jnp
from jax import lax
from jax.experimental import pallas as pl
from jax.experimental.pallas import tpu as pltpu
from jax.experimental.pallas import tpu_sc as plsc

N = 10000
E = 320000
DF = 128
DH = 256
DC = 16

NC = 2
NS = 16
NW = NC * NS
EW = E // NW
RT = N // NS


def _make_sc_agg(D, C, S, count, dtype=jnp.float32):
  NB = 4
  NSC = EW // (S * C)
  assert NSC * S * C == EW and C <= 128 and S % NB == 0 and S > NB
  mesh = plsc.VectorSubcoreMesh(core_axis_name="c", subcore_axis_name="s")

  out_type = [jax.ShapeDtypeStruct((NC, N, D), dtype)]
  scratch = [
      pltpu.VMEM((S, C), jnp.int32),
      pltpu.VMEM((S, C), jnp.int32),
      [pltpu.VMEM((C, D), dtype) for _ in range(NB)],
      [pltpu.SemaphoreType.DMA for _ in range(NB)],
      [pltpu.SemaphoreType.DMA for _ in range(NB)],
  ]
  scratch.append(pltpu.VMEM_SHARED((N, D), dtype))
  if count:
    out_type.append(jax.ShapeDtypeStruct((NC, N, DC), jnp.float32))
    scratch += [
        pltpu.VMEM((C, DC), jnp.float32),
        pltpu.VMEM_SHARED((N, DC), jnp.float32),
        pltpu.SemaphoreType.DMA,
    ]

  @functools.partial(pl.kernel,
                     out_type=out_type if count else out_type[0], mesh=mesh,
                     compiler_params=pltpu.CompilerParams(
                         use_tc_tiling_on_sc=False),
                     scratch_types=scratch)
  def sc_agg(table, srcr, dstr, zrow, *rest):
    if count:
      (zcnt, out, outc, isrc, idst, rows, gsem, ssem, acc,
       ones, cacc, osem) = rest
    else:
      out, isrc, idst, rows, gsem, ssem, acc = rest
    c = lax.axis_index("c")
    s = lax.axis_index("s")
    w = s * NC + c
    base = s * RT

    pltpu.sync_copy(zrow, acc.at[pl.ds(base, RT)])
    if count:
      pltpu.sync_copy(zcnt, cacc.at[pl.ds(base, RT)])

      @pl.loop(0, C)
      def _fill(i):
        ones[i] = jnp.ones((DC,), jnp.float32)

    plsc.subcore_barrier()

    @pl.loop(0, NSC)
    def _super(u):
      pltpu.sync_copy(srcr.at[w].at[u], isrc)
      pltpu.sync_copy(dstr.at[w].at[u], idst)
      for b in range(NB - 1):
        pltpu.async_copy(table.at[isrc.at[b]], rows[b], gsem[b])

      @pl.loop(0, S, step=NB)
      def _slots(k0):
        for j in range(NB):
          k = k0 + j
          b = j
          bp = (j + NB - 1) % NB
          pltpu.make_async_copy(table.at[isrc.at[k]], rows[b], gsem[b]).wait()
          pltpu.async_copy(rows[b], acc.at[idst.at[k]], ssem[b], add=True)
          if count:
            pltpu.async_copy(ones, cacc.at[idst.at[k]], osem, add=True)

          @pl.when(k >= 1)
          def _():
            pltpu.make_async_copy(rows[bp], acc.at[idst.at[0]],
                                  ssem[bp]).wait()

          @pl.when(k + NB - 1 < S)
          def _():
            pltpu.async_copy(table.at[isrc.at[k + NB - 1]], rows[bp],
                             gsem[bp])

      pltpu.make_async_copy(rows[NB - 1], acc.at[idst.at[0]],
                            ssem[NB - 1]).wait()
      if count:
        @pl.loop(0, S)
        def _drain(i):
          pltpu.make_async_copy(ones, cacc.at[idst.at[0]], osem).wait()

    plsc.subcore_barrier()

    pltpu.sync_copy(acc.at[pl.ds(base, RT)], out.at[c].at[pl.ds(base, RT)])
    if count:
      pltpu.sync_copy(cacc.at[pl.ds(base, RT)], outc.at[c].at[pl.ds(base, RT)])

  return sc_agg


CA, SA = 125, 16
CF, SF = 125, 16
_sc_agg_c = _make_sc_agg(DF, CA, SA, True, jnp.bfloat16)
_sc_agg_f = _make_sc_agg(DF, CF, SF, False, jnp.bfloat16)

BN = 1000
GRID = N // BN


def _tc1_body(aggp, cntp, x, w1l, w1r, b1, w2l, w2r, b2, g, hr, inv):
  agg = (aggp[0].astype(jnp.float32) + aggp[1].astype(jnp.float32))
  cnt = (cntp[0] + cntp[1])[:, :1]
  iv = 1.0 / jnp.maximum(cnt, 1.0)
  mean = agg * iv
  h = (jnp.dot(mean, w1l[...], preferred_element_type=jnp.float32)
       + jnp.dot(x[...], w1r[...], preferred_element_type=jnp.float32)
       + b1[...])
  h = jnp.maximum(h, 0.0)
  g[...] = jnp.dot(h, w2l[...],
                   preferred_element_type=jnp.float32).astype(jnp.bfloat16)
  hr[...] = (jnp.dot(h, w2r[...], preferred_element_type=jnp.float32)
             + b2[...])
  inv[...] = iv


_tc1 = pl.pallas_call(
    _tc1_body,
    grid=(GRID,),
    in_specs=[
        pl.BlockSpec((NC, BN, DF), lambda i: (0, i, 0)),
        pl.BlockSpec((NC, BN, DC), lambda i: (0, i, 0)),
        pl.BlockSpec((BN, DF), lambda i: (i, 0)),
        pl.BlockSpec((DF, DH), lambda i: (0, 0)),
        pl.BlockSpec((DF, DH), lambda i: (0, 0)),
        pl.BlockSpec((1, DH), lambda i: (0, 0)),
        pl.BlockSpec((DH, DF), lambda i: (0, 0)),
        pl.BlockSpec((DH, DF), lambda i: (0, 0)),
        pl.BlockSpec((1, DF), lambda i: (0, 0)),
    ],
    out_specs=[
        pl.BlockSpec((BN, DF), lambda i: (i, 0)),
        pl.BlockSpec((BN, DF), lambda i: (i, 0)),
        pl.BlockSpec((BN, 1), lambda i: (i, 0)),
    ],
    out_shape=[
        jax.ShapeDtypeStruct((N, DF), jnp.bfloat16),
        jax.ShapeDtypeStruct((N, DF), jnp.float32),
        jax.ShapeDtypeStruct((N, 1), jnp.float32),
    ],
)


def _tc2_body(agg2, inv, hr, out):
  s = agg2[0].astype(jnp.float32) + agg2[1].astype(jnp.float32)
  out[...] = s * inv[...] + hr[...]


_tc2 = pl.pallas_call(
    _tc2_body,
    grid=(GRID,),
    in_specs=[
        pl.BlockSpec((NC, BN, DF), lambda i: (0, i, 0)),
        pl.BlockSpec((BN, 1), lambda i: (i, 0)),
        pl.BlockSpec((BN, DF), lambda i: (i, 0)),
    ],
    out_specs=pl.BlockSpec((BN, DF), lambda i: (i, 0)),
    out_shape=jax.ShapeDtypeStruct((N, DF), jnp.float32),
)


def kernel(x, edge_index, W1_l, W1_r, b1, W2_l, W2_r, b2):
  src = edge_index[0].astype(jnp.int32)
  dst = edge_index[1].astype(jnp.int32)
  src_a = src.reshape(NW, EW // (SA * CA), SA, CA)
  dst_a = dst.reshape(NW, EW // (SA * CA), SA, CA)
  src_f = src.reshape(NW, EW // (SF * CF), SF, CF)
  dst_f = dst.reshape(NW, EW // (SF * CF), SF, CF)
  zrow = jnp.zeros((RT, DF), jnp.bfloat16)
  zcnt = jnp.zeros((RT, DC), jnp.float32)
  xb = x.astype(jnp.bfloat16)

  aggp, cntp = _sc_agg_c(xb, src_a, dst_a, zrow, zcnt)
  g, hr, inv = _tc1(aggp, cntp, x, W1_l, W1_r, b1.reshape(1, DH),
                    W2_l, W2_r, b2.reshape(1, DF))
  agg2 = _sc_agg_f(g, src_f, dst_f, zrow)
  return _tc2(agg2, inv, hr)
